# Initial kernel scaffold; baseline (speedup 1.0000x reference)
import jax
import jax.numpy as jnp
from jax.experimental import pallas as pl


def kernel(x, edge_index, W, b):
    raise NotImplementedError("write your pallas kernel here")


# speedup vs baseline: 14.6103x; 14.6103x over previous
import functools

import jax
import jax.numpy as jnp
from jax import lax
from jax.experimental import pallas as pl
from jax.experimental.pallas import tpu as pltpu
from jax.experimental.pallas import tpu_sc as plsc

N = 10000
E = 160000
D = 256
H = D // 2
NC, NS = 2, 16
NPAD = 10240
RPAD = NPAD // NS
RTA = 632
RTB = N - 15 * RTA

_MESH = dict(core_axis_name="c", subcore_axis_name="s")

DEG_CHUNK = 40
DEG_EPW = E // (NC * NS)
DEG_NCH = DEG_EPW // DEG_CHUNK


@functools.partial(
    pl.kernel,
    out_type=jax.ShapeDtypeStruct((NC, NPAD), jnp.float32),
    mesh=plsc.VectorSubcoreMesh(**_MESH),
    scratch_types=[
        pltpu.VMEM((DEG_NCH, DEG_CHUNK), jnp.int32),
        pltpu.VMEM((48,), jnp.float32),
        pltpu.VMEM((RPAD,), jnp.float32),
        pltpu.VMEM_SHARED((NPAD,), jnp.float32),
    ],
)
def _deg_kernel(dst4, degp, dst_v, ones_v, zrow_v, deg_sh):
    c = lax.axis_index("c")
    s = lax.axis_index("s")
    for i in range(RPAD // 16):
        zrow_v[pl.ds(i * 16, 16)] = jnp.zeros((16,), jnp.float32)
    for i in range(3):
        ones_v[pl.ds(i * 16, 16)] = jnp.ones((16,), jnp.float32)

    pltpu.sync_copy(zrow_v, deg_sh.at[pl.ds(s * RPAD, RPAD)])
    plsc.subcore_barrier()
    pltpu.sync_copy(dst4.at[c, s], dst_v)

    def step(j, carry):
        pltpu.sync_copy(
            ones_v.at[pl.ds(0, DEG_CHUNK)],
            deg_sh.at[dst_v.at[j]],
            add=True,
        )
        return carry

    lax.fori_loop(0, DEG_NCH, step, 0)
    plsc.subcore_barrier()
    pltpu.sync_copy(
        deg_sh.at[pl.ds(s * RPAD, RPAD)],
        degp.at[c, pl.ds(s * RPAD, RPAD)],
    )


AGG_CHUNK = 80
AGG_EPT = E // NS
AGG_NCH = AGG_EPT // AGG_CHUNK


@functools.partial(
    pl.kernel,
    out_type=(
        jax.ShapeDtypeStruct((N, H), jnp.float32),
        jax.ShapeDtypeStruct((N, H), jnp.float32),
    ),
    mesh=plsc.VectorSubcoreMesh(**_MESH),
    scratch_types=[
        pltpu.VMEM((AGG_NCH, AGG_CHUNK), jnp.int32),
        pltpu.VMEM((AGG_NCH, AGG_CHUNK), jnp.int32),
        pltpu.VMEM((AGG_CHUNK, H), jnp.float32),
        pltpu.VMEM_SHARED((N, H), jnp.float32),
        pltpu.SemaphoreType.DMA,
    ],
)
def _agg_kernel(src3, dst3, y0, y1, o0, o1, src_v, dst_v, rows_v, acc_sh, gsem):
    c = lax.axis_index("c")
    s = lax.axis_index("s")
    pltpu.sync_copy(src3.at[s], src_v)
    pltpu.sync_copy(dst3.at[s], dst_v)
    r0 = s * RTA

    def rows_copy(get_src, get_dst):
        pltpu.sync_copy(get_src(r0, RTB), get_dst(r0, RTB))

        @pl.when(s < NS - 1)
        def _():
            pltpu.sync_copy(get_src(r0 + RTB, RTA - RTB),
                            get_dst(r0 + RTB, RTA - RTB))

    def init_for(yref):
        def f():
            rows_copy(lambda o, n: yref.at[pl.ds(o, n)],
                      lambda o, n: acc_sh.at[pl.ds(o, n)])
        return f

    pl.when(c == 0)(init_for(y0))
    pl.when(c == 1)(init_for(y1))
    plsc.subcore_barrier()

    def loop_for(yref):
        def f():
            def step(j, carry):
                pltpu.async_copy(yref.at[src_v.at[j]], rows_v, gsem).wait()
                pltpu.sync_copy(rows_v, acc_sh.at[dst_v.at[j]], add=True)
                return carry

            lax.fori_loop(0, AGG_NCH, step, 0)
        return f

    pl.when(c == 0)(loop_for(y0))
    pl.when(c == 1)(loop_for(y1))
    plsc.subcore_barrier()

    def out_for(oref):
        def f():
            rows_copy(lambda o, n: acc_sh.at[pl.ds(o, n)],
                      lambda o, n: oref.at[pl.ds(o, n)])
        return f

    pl.when(c == 0)(out_for(o0))
    pl.when(c == 1)(out_for(o1))


BM = 400


def _mm_body(x_ref, w_ref, degt_ref, y0_ref, y1_ref):
    dis = lax.rsqrt(degt_ref[:, 0:1] + degt_ref[:, 1:2] + 1.0)
    xw = jnp.dot(x_ref[...], w_ref[...], preferred_element_type=jnp.float32)
    y = xw * dis
    y0_ref[...] = y[:, :H]
    y1_ref[...] = y[:, H:]


_mm_call = pl.pallas_call(
    _mm_body,
    grid=(N // BM,),
    in_specs=[
        pl.BlockSpec((BM, D), lambda i: (i, 0)),
        pl.BlockSpec((D, D), lambda i: (0, 0)),
        pl.BlockSpec((BM, 2), lambda i: (i, 0)),
    ],
    out_specs=(
        pl.BlockSpec((BM, H), lambda i: (i, 0)),
        pl.BlockSpec((BM, H), lambda i: (i, 0)),
    ),
    out_shape=(
        jax.ShapeDtypeStruct((N, H), jnp.float32),
        jax.ShapeDtypeStruct((N, H), jnp.float32),
    ),
)


def _epi_body(a0_ref, a1_ref, degt_ref, x_ref, b_ref, o_ref):
    dis = lax.rsqrt(degt_ref[:, 0:1] + degt_ref[:, 1:2] + 1.0)
    agg = jnp.concatenate([a0_ref[...], a1_ref[...]], axis=1)
    o_ref[...] = jnp.maximum(agg * dis + b_ref[...] + x_ref[...], 0.0)


_epi_call = pl.pallas_call(
    _epi_body,
    grid=(N // BM,),
    in_specs=[
        pl.BlockSpec((BM, H), lambda i: (i, 0)),
        pl.BlockSpec((BM, H), lambda i: (i, 0)),
        pl.BlockSpec((BM, 2), lambda i: (i, 0)),
        pl.BlockSpec((BM, D), lambda i: (i, 0)),
        pl.BlockSpec((1, D), lambda i: (0, 0)),
    ],
    out_specs=pl.BlockSpec((BM, D), lambda i: (i, 0)),
    out_shape=jax.ShapeDtypeStruct((N, D), jnp.float32),
)


def kernel(x, edge_index, W, b):
    src = edge_index[0].astype(jnp.int32)
    dst = edge_index[1].astype(jnp.int32)

    dst4 = dst.reshape(NC, NS, DEG_NCH, DEG_CHUNK)
    degp = _deg_kernel(dst4)
    degt = degp.T[:N]

    y0, y1 = _mm_call(x, W, degt)

    src3 = src.reshape(NS, AGG_NCH, AGG_CHUNK)
    dst3 = dst.reshape(NS, AGG_NCH, AGG_CHUNK)
    a0, a1 = _agg_kernel(src3, dst3, y0, y1)

    return _epi_call(a0, a1, degt, x, b.reshape(1, D))


# speedup vs baseline: 18.5345x; 1.2686x over previous
import functools

import jax
import jax.numpy as jnp
from jax import lax
from jax.experimental import pallas as pl
from jax.experimental.pallas import tpu as pltpu
from jax.experimental.pallas import tpu_sc as plsc

N = 10000
E = 160000
D = 256
H = D // 2
NC, NS = 2, 16
NPAD = 10240
RPAD = NPAD // NS
RTA = 632
RTB = N - 15 * RTA

_MESH = dict(core_axis_name="c", subcore_axis_name="s")

DEG_CHUNK = 125
DEG_EPW = E // (NC * NS)
DEG_NCH = DEG_EPW // DEG_CHUNK


@functools.partial(
    pl.kernel,
    out_type=jax.ShapeDtypeStruct((NC, NPAD), jnp.float32),
    mesh=plsc.VectorSubcoreMesh(**_MESH),
    scratch_types=[
        pltpu.VMEM((8, DEG_CHUNK), jnp.int32),
        pltpu.VMEM((128,), jnp.float32),
        pltpu.VMEM((RPAD,), jnp.float32),
        pltpu.VMEM_SHARED((NPAD,), jnp.float32),
    ],
)
def _deg_kernel(dst4, degp, dst_v, ones_v, zrow_v, deg_sh):
    c = lax.axis_index("c")
    s = lax.axis_index("s")
    for i in range(RPAD // 16):
        zrow_v[pl.ds(i * 16, 16)] = jnp.zeros((16,), jnp.float32)
    for i in range(8):
        ones_v[pl.ds(i * 16, 16)] = jnp.ones((16,), jnp.float32)

    pltpu.sync_copy(zrow_v, deg_sh.at[pl.ds(s * RPAD, RPAD)])
    plsc.subcore_barrier()

    for w in range(DEG_NCH // 8):
        pltpu.sync_copy(dst4.at[c, s, pl.ds(w * 8, 8)], dst_v)

        def step(j, carry):
            pltpu.sync_copy(
                ones_v.at[pl.ds(0, DEG_CHUNK)],
                deg_sh.at[dst_v.at[j]],
                add=True,
            )
            return carry

        lax.fori_loop(0, 8, step, 0)
    plsc.subcore_barrier()
    pltpu.sync_copy(
        deg_sh.at[pl.ds(s * RPAD, RPAD)],
        degp.at[c, pl.ds(s * RPAD, RPAD)],
    )


AGG_CHUNK = 80
AGG_EPT = E // NS
AGG_NCH = AGG_EPT // AGG_CHUNK


@functools.partial(
    pl.kernel,
    out_type=(
        jax.ShapeDtypeStruct((N, H), jnp.float32),
        jax.ShapeDtypeStruct((N, H), jnp.float32),
    ),
    mesh=plsc.VectorSubcoreMesh(**_MESH),
    scratch_types=[
        pltpu.VMEM((2, AGG_CHUNK), jnp.int32),
        pltpu.VMEM((2, AGG_CHUNK), jnp.int32),
        pltpu.VMEM((AGG_CHUNK, H), jnp.float32),
        pltpu.VMEM((AGG_CHUNK, H), jnp.float32),
        pltpu.VMEM_SHARED((N, H), jnp.float32),
        pltpu.SemaphoreType.DMA,
        pltpu.SemaphoreType.DMA,
        pltpu.SemaphoreType.DMA,
        pltpu.SemaphoreType.DMA,
    ],
)
def _agg_kernel(eidx, y0, y1, o0, o1, idx_a, idx_b, rows_a, rows_b,
                acc_sh, isem_a, isem_b, gsem_a, gsem_b):
    c = lax.axis_index("c")
    s = lax.axis_index("s")
    r0 = s * RTA

    def rows_copy(get_src, get_dst):
        pltpu.sync_copy(get_src(r0, RTB), get_dst(r0, RTB))

        @pl.when(s < NS - 1)
        def _():
            pltpu.sync_copy(get_src(r0 + RTB, RTA - RTB),
                            get_dst(r0 + RTB, RTA - RTB))

    def init_for(yref):
        def f():
            rows_copy(lambda o, n: yref.at[pl.ds(o, n)],
                      lambda o, n: acc_sh.at[pl.ds(o, n)])
        return f

    pl.when(c == 0)(init_for(y0))
    pl.when(c == 1)(init_for(y1))
    plsc.subcore_barrier()

    def loop_for(yref):
        def f():
            def fire_idx(cj, ibuf, isem):
                pltpu.async_copy(eidx.at[s, cj], ibuf, isem)

            def drain_idx(ibuf, isem):
                pltpu.make_async_copy(eidx.at[s, 0], ibuf, isem).wait()

            def fire_rows(ibuf, buf, gsem):
                pltpu.async_copy(yref.at[ibuf.at[0]], buf, gsem)

            def drain_rows(ibuf, buf, gsem):
                pltpu.make_async_copy(yref.at[ibuf.at[0]], buf, gsem).wait()

            def scat(ibuf, buf):
                pltpu.sync_copy(buf, acc_sh.at[ibuf.at[1]], add=True)

            fire_idx(0, idx_a, isem_a)
            drain_idx(idx_a, isem_a)
            fire_rows(idx_a, rows_a, gsem_a)
            fire_idx(1, idx_b, isem_b)

            @pl.loop(1, AGG_NCH, step=2)
            def _(j):
                drain_idx(idx_b, isem_b)
                fire_rows(idx_b, rows_b, gsem_b)
                drain_rows(idx_a, rows_a, gsem_a)
                scat(idx_a, rows_a)
                fire_idx(j + 1, idx_a, isem_a)
                drain_idx(idx_a, isem_a)
                fire_rows(idx_a, rows_a, gsem_a)
                drain_rows(idx_b, rows_b, gsem_b)
                scat(idx_b, rows_b)

                @pl.when(j + 2 < AGG_NCH)
                def _():
                    fire_idx(j + 2, idx_b, isem_b)

            drain_rows(idx_a, rows_a, gsem_a)
            scat(idx_a, rows_a)
        return f

    pl.when(c == 0)(loop_for(y0))
    pl.when(c == 1)(loop_for(y1))
    plsc.subcore_barrier()

    def out_for(oref):
        def f():
            rows_copy(lambda o, n: acc_sh.at[pl.ds(o, n)],
                      lambda o, n: oref.at[pl.ds(o, n)])
        return f

    pl.when(c == 0)(out_for(o0))
    pl.when(c == 1)(out_for(o1))


BM = 400


def _mm_body(x_ref, w_ref, degt_ref, y0_ref, y1_ref):
    dis = lax.rsqrt(degt_ref[:, 0:1] + degt_ref[:, 1:2] + 1.0)
    xw = jnp.dot(x_ref[...], w_ref[...], preferred_element_type=jnp.float32)
    y = xw * dis
    y0_ref[...] = y[:, :H]
    y1_ref[...] = y[:, H:]


_mm_call = pl.pallas_call(
    _mm_body,
    grid=(N // BM,),
    in_specs=[
        pl.BlockSpec((BM, D), lambda i: (i, 0)),
        pl.BlockSpec((D, D), lambda i: (0, 0)),
        pl.BlockSpec((BM, 2), lambda i: (i, 0)),
    ],
    out_specs=(
        pl.BlockSpec((BM, H), lambda i: (i, 0)),
        pl.BlockSpec((BM, H), lambda i: (i, 0)),
    ),
    out_shape=(
        jax.ShapeDtypeStruct((N, H), jnp.float32),
        jax.ShapeDtypeStruct((N, H), jnp.float32),
    ),
)


def _epi_body(a0_ref, a1_ref, degt_ref, x_ref, b_ref, o_ref):
    dis = lax.rsqrt(degt_ref[:, 0:1] + degt_ref[:, 1:2] + 1.0)
    agg = jnp.concatenate([a0_ref[...], a1_ref[...]], axis=1)
    o_ref[...] = jnp.maximum(agg * dis + b_ref[...] + x_ref[...], 0.0)


_epi_call = pl.pallas_call(
    _epi_body,
    grid=(N // BM,),
    in_specs=[
        pl.BlockSpec((BM, H), lambda i: (i, 0)),
        pl.BlockSpec((BM, H), lambda i: (i, 0)),
        pl.BlockSpec((BM, 2), lambda i: (i, 0)),
        pl.BlockSpec((BM, D), lambda i: (i, 0)),
        pl.BlockSpec((1, D), lambda i: (0, 0)),
    ],
    out_specs=pl.BlockSpec((BM, D), lambda i: (i, 0)),
    out_shape=jax.ShapeDtypeStruct((N, D), jnp.float32),
)


def kernel(x, edge_index, W, b):
    src = edge_index[0].astype(jnp.int32)
    dst = edge_index[1].astype(jnp.int32)

    dst4 = dst.reshape(NC, NS, DEG_NCH, DEG_CHUNK)
    degp = _deg_kernel(dst4)
    degt = degp.T[:N]

    y0, y1 = _mm_call(x, W, degt)

    src3 = src.reshape(NS, AGG_NCH, AGG_CHUNK)
    dst3 = dst.reshape(NS, AGG_NCH, AGG_CHUNK)
    eidx = jnp.stack([src3, dst3], axis=2)
    a0, a1 = _agg_kernel(eidx, y0, y1)

    return _epi_call(a0, a1, degt, x, b.reshape(1, D))


# speedup vs baseline: 20.8405x; 1.1244x over previous
import functools

import jax
import jax.numpy as jnp
from jax import lax
from jax.experimental import pallas as pl
from jax.experimental.pallas import tpu as pltpu
from jax.experimental.pallas import tpu_sc as plsc

N = 10000
E = 160000
D = 256
H = D // 2
NC, NS = 2, 16
NPAD = 10240
RPAD = NPAD // NS
RTA = 632
RTB = N - 15 * RTA

_MESH = dict(core_axis_name="c", subcore_axis_name="s")

DEG_CHUNK = 125
DEG_EPW = E // (NC * NS)
DEG_NCH = DEG_EPW // DEG_CHUNK


@functools.partial(
    pl.kernel,
    out_type=jax.ShapeDtypeStruct((NC, NPAD), jnp.float32),
    mesh=plsc.VectorSubcoreMesh(**_MESH),
    scratch_types=[
        pltpu.VMEM((8, DEG_CHUNK), jnp.int32),
        pltpu.VMEM((128,), jnp.float32),
        pltpu.VMEM((RPAD,), jnp.float32),
        pltpu.VMEM_SHARED((NPAD,), jnp.float32),
    ],
)
def _deg_kernel(dst4, degp, dst_v, ones_v, zrow_v, deg_sh):
    c = lax.axis_index("c")
    s = lax.axis_index("s")
    for i in range(RPAD // 16):
        zrow_v[pl.ds(i * 16, 16)] = jnp.zeros((16,), jnp.float32)
    for i in range(8):
        ones_v[pl.ds(i * 16, 16)] = jnp.ones((16,), jnp.float32)

    pltpu.sync_copy(zrow_v, deg_sh.at[pl.ds(s * RPAD, RPAD)])
    plsc.subcore_barrier()

    for w in range(DEG_NCH // 8):
        pltpu.sync_copy(dst4.at[c, s, pl.ds(w * 8, 8)], dst_v)

        def step(j, carry):
            pltpu.sync_copy(
                ones_v.at[pl.ds(0, DEG_CHUNK)],
                deg_sh.at[dst_v.at[j]],
                add=True,
            )
            return carry

        lax.fori_loop(0, 8, step, 0)
    plsc.subcore_barrier()
    pltpu.sync_copy(
        deg_sh.at[pl.ds(s * RPAD, RPAD)],
        degp.at[c, pl.ds(s * RPAD, RPAD)],
    )


AGG_CHUNK = 125
AGG_EPT = E // NS
AGG_NCH = AGG_EPT // AGG_CHUNK


@functools.partial(
    pl.kernel,
    out_type=(
        jax.ShapeDtypeStruct((N, H), jnp.float32),
        jax.ShapeDtypeStruct((N, H), jnp.float32),
    ),
    mesh=plsc.VectorSubcoreMesh(**_MESH),
    scratch_types=[
        pltpu.VMEM((2, AGG_CHUNK), jnp.int32),
        pltpu.VMEM((2, AGG_CHUNK), jnp.int32),
        pltpu.VMEM((AGG_CHUNK, H), jnp.float32),
        pltpu.VMEM((AGG_CHUNK, H), jnp.float32),
        pltpu.VMEM_SHARED((N, H), jnp.float32),
        pltpu.SemaphoreType.DMA,
        pltpu.SemaphoreType.DMA,
        pltpu.SemaphoreType.DMA,
        pltpu.SemaphoreType.DMA,
    ],
)
def _agg_kernel(eidx, y0, y1, o0, o1, idx_a, idx_b, rows_a, rows_b,
                acc_sh, isem_a, isem_b, gsem_a, gsem_b):
    c = lax.axis_index("c")
    s = lax.axis_index("s")
    r0 = s * RTA

    def rows_copy(get_src, get_dst):
        pltpu.sync_copy(get_src(r0, RTB), get_dst(r0, RTB))

        @pl.when(s < NS - 1)
        def _():
            pltpu.sync_copy(get_src(r0 + RTB, RTA - RTB),
                            get_dst(r0 + RTB, RTA - RTB))

    def init_for(yref):
        def f():
            rows_copy(lambda o, n: yref.at[pl.ds(o, n)],
                      lambda o, n: acc_sh.at[pl.ds(o, n)])
        return f

    pl.when(c == 0)(init_for(y0))
    pl.when(c == 1)(init_for(y1))
    plsc.subcore_barrier()

    def loop_for(yref):
        def f():
            def fire_idx(cj, ibuf, isem):
                pltpu.async_copy(eidx.at[s, cj], ibuf, isem)

            def drain_idx(ibuf, isem):
                pltpu.make_async_copy(eidx.at[s, 0], ibuf, isem).wait()

            def fire_rows(ibuf, buf, gsem):
                pltpu.async_copy(yref.at[ibuf.at[0]], buf, gsem)

            def drain_rows(ibuf, buf, gsem):
                pltpu.make_async_copy(yref.at[ibuf.at[0]], buf, gsem).wait()

            def scat(ibuf, buf):
                pltpu.sync_copy(buf, acc_sh.at[ibuf.at[1]], add=True)

            fire_idx(0, idx_a, isem_a)
            drain_idx(idx_a, isem_a)
            fire_rows(idx_a, rows_a, gsem_a)
            fire_idx(1, idx_b, isem_b)

            @pl.loop(1, AGG_NCH, step=2)
            def _(j):
                drain_idx(idx_b, isem_b)
                fire_rows(idx_b, rows_b, gsem_b)
                drain_rows(idx_a, rows_a, gsem_a)
                scat(idx_a, rows_a)

                @pl.when(j + 1 < AGG_NCH)
                def _():
                    fire_idx(j + 1, idx_a, isem_a)
                    drain_idx(idx_a, isem_a)
                    fire_rows(idx_a, rows_a, gsem_a)

                drain_rows(idx_b, rows_b, gsem_b)
                scat(idx_b, rows_b)

                @pl.when(j + 2 < AGG_NCH)
                def _():
                    fire_idx(j + 2, idx_b, isem_b)
        return f

    pl.when(c == 0)(loop_for(y0))
    pl.when(c == 1)(loop_for(y1))
    plsc.subcore_barrier()

    def out_for(oref):
        def f():
            rows_copy(lambda o, n: acc_sh.at[pl.ds(o, n)],
                      lambda o, n: oref.at[pl.ds(o, n)])
        return f

    pl.when(c == 0)(out_for(o0))
    pl.when(c == 1)(out_for(o1))


BM = 400


def _mm_body(x_ref, w_ref, degt_ref, y0_ref, y1_ref):
    dis = lax.rsqrt(degt_ref[:, 0:1] + degt_ref[:, 1:2] + 1.0)
    xw = jnp.dot(x_ref[...], w_ref[...], preferred_element_type=jnp.float32)
    y = xw * dis
    y0_ref[...] = y[:, :H]
    y1_ref[...] = y[:, H:]


_mm_call = pl.pallas_call(
    _mm_body,
    grid=(N // BM,),
    in_specs=[
        pl.BlockSpec((BM, D), lambda i: (i, 0)),
        pl.BlockSpec((D, D), lambda i: (0, 0)),
        pl.BlockSpec((BM, 2), lambda i: (i, 0)),
    ],
    out_specs=(
        pl.BlockSpec((BM, H), lambda i: (i, 0)),
        pl.BlockSpec((BM, H), lambda i: (i, 0)),
    ),
    out_shape=(
        jax.ShapeDtypeStruct((N, H), jnp.float32),
        jax.ShapeDtypeStruct((N, H), jnp.float32),
    ),
)


def _epi_body(a0_ref, a1_ref, degt_ref, x_ref, b_ref, o_ref):
    dis = lax.rsqrt(degt_ref[:, 0:1] + degt_ref[:, 1:2] + 1.0)
    agg = jnp.concatenate([a0_ref[...], a1_ref[...]], axis=1)
    o_ref[...] = jnp.maximum(agg * dis + b_ref[...] + x_ref[...], 0.0)


_epi_call = pl.pallas_call(
    _epi_body,
    grid=(N // BM,),
    in_specs=[
        pl.BlockSpec((BM, H), lambda i: (i, 0)),
        pl.BlockSpec((BM, H), lambda i: (i, 0)),
        pl.BlockSpec((BM, 2), lambda i: (i, 0)),
        pl.BlockSpec((BM, D), lambda i: (i, 0)),
        pl.BlockSpec((1, D), lambda i: (0, 0)),
    ],
    out_specs=pl.BlockSpec((BM, D), lambda i: (i, 0)),
    out_shape=jax.ShapeDtypeStruct((N, D), jnp.float32),
)


def kernel(x, edge_index, W, b):
    src = edge_index[0].astype(jnp.int32)
    dst = edge_index[1].astype(jnp.int32)

    dst4 = dst.reshape(NC, NS, DEG_NCH, DEG_CHUNK)
    degp = _deg_kernel(dst4)
    degt = degp.T[:N]

    y0, y1 = _mm_call(x, W, degt)

    src3 = src.reshape(NS, AGG_NCH, AGG_CHUNK)
    dst3 = dst.reshape(NS, AGG_NCH, AGG_CHUNK)
    eidx = jnp.stack([src3, dst3], axis=2)
    a0, a1 = _agg_kernel(eidx, y0, y1)

    return _epi_call(a0, a1, degt, x, b.reshape(1, D))


# speedup vs baseline: 23.2871x; 1.1174x over previous
import functools

import jax
import jax.numpy as jnp
from jax import lax
from jax.experimental import pallas as pl
from jax.experimental.pallas import tpu as pltpu
from jax.experimental.pallas import tpu_sc as plsc

N = 10000
E = 160000
D = 256
H = D // 2
NC, NS = 2, 16
NPAD = 10240
RPAD = NPAD // NS
RTA = 632
RTB = N - 15 * RTA

_MESH = dict(core_axis_name="c", subcore_axis_name="s")

DEG_CHUNK = 125
DEG_EPW = E // (NC * NS)
DEG_NCH = DEG_EPW // DEG_CHUNK


@functools.partial(
    pl.kernel,
    out_type=jax.ShapeDtypeStruct((NC, NPAD), jnp.float32),
    mesh=plsc.VectorSubcoreMesh(**_MESH),
    scratch_types=[
        pltpu.VMEM((8, DEG_CHUNK), jnp.int32),
        pltpu.VMEM((128,), jnp.float32),
        pltpu.VMEM((RPAD,), jnp.float32),
        pltpu.VMEM_SHARED((NPAD,), jnp.float32),
    ],
)
def _deg_kernel(dst4, degp, dst_v, ones_v, zrow_v, deg_sh):
    c = lax.axis_index("c")
    s = lax.axis_index("s")
    for i in range(RPAD // 16):
        zrow_v[pl.ds(i * 16, 16)] = jnp.zeros((16,), jnp.float32)
    for i in range(8):
        ones_v[pl.ds(i * 16, 16)] = jnp.ones((16,), jnp.float32)

    pltpu.sync_copy(zrow_v, deg_sh.at[pl.ds(s * RPAD, RPAD)])
    plsc.subcore_barrier()

    for w in range(DEG_NCH // 8):
        pltpu.sync_copy(dst4.at[c, s, pl.ds(w * 8, 8)], dst_v)

        def step(j, carry):
            pltpu.sync_copy(
                ones_v.at[pl.ds(0, DEG_CHUNK)],
                deg_sh.at[dst_v.at[j]],
                add=True,
            )
            return carry

        lax.fori_loop(0, 8, step, 0)
    plsc.subcore_barrier()
    pltpu.sync_copy(
        deg_sh.at[pl.ds(s * RPAD, RPAD)],
        degp.at[c, pl.ds(s * RPAD, RPAD)],
    )


AGG_CHUNK = 80
AGG_EPT = E // NS
AGG_NCH = AGG_EPT // AGG_CHUNK
NROWS = 3
NIDX = 4


@functools.partial(
    pl.kernel,
    out_type=(
        jax.ShapeDtypeStruct((N, H), jnp.float32),
        jax.ShapeDtypeStruct((N, H), jnp.float32),
    ),
    mesh=plsc.VectorSubcoreMesh(**_MESH),
    scratch_types=(
        [pltpu.VMEM((2, AGG_CHUNK), jnp.int32) for _ in range(NIDX)]
        + [pltpu.VMEM((AGG_CHUNK, H), jnp.float32) for _ in range(NROWS)]
        + [pltpu.VMEM_SHARED((N, H), jnp.float32)]
        + [pltpu.SemaphoreType.DMA] * (NIDX + 2 * NROWS)
    ),
)
def _agg_kernel(eidx, y0, y1, o0, o1, *refs):
    ibufs = refs[:NIDX]
    rbufs = refs[NIDX:NIDX + NROWS]
    acc_sh = refs[NIDX + NROWS]
    isems = refs[NIDX + NROWS + 1:NIDX + NROWS + 1 + NIDX]
    gsems = refs[NIDX + NROWS + 1 + NIDX:NIDX + NROWS + 1 + NIDX + NROWS]
    ssems = refs[NIDX + NROWS + 1 + NIDX + NROWS:]
    c = lax.axis_index("c")
    s = lax.axis_index("s")
    r0 = s * RTA

    def rows_copy(get_src, get_dst):
        pltpu.sync_copy(get_src(r0, RTB), get_dst(r0, RTB))

        @pl.when(s < NS - 1)
        def _():
            pltpu.sync_copy(get_src(r0 + RTB, RTA - RTB),
                            get_dst(r0 + RTB, RTA - RTB))

    def init_for(yref):
        def f():
            rows_copy(lambda o, n: yref.at[pl.ds(o, n)],
                      lambda o, n: acc_sh.at[pl.ds(o, n)])
        return f

    pl.when(c == 0)(init_for(y0))
    pl.when(c == 1)(init_for(y1))
    plsc.subcore_barrier()

    def loop_for(yref):
        def f():
            def fire_idx(j):
                pltpu.async_copy(eidx.at[s, j], ibufs[j % NIDX],
                                 isems[j % NIDX])

            def drain_idx(j):
                pltpu.make_async_copy(eidx.at[s, 0], ibufs[j % NIDX],
                                      isems[j % NIDX]).wait()

            def fire_rows(j):
                pltpu.async_copy(yref.at[ibufs[j % NIDX].at[0]],
                                 rbufs[j % NROWS], gsems[j % NROWS])

            def drain_rows(j):
                pltpu.make_async_copy(yref.at[ibufs[j % NIDX].at[0]],
                                      rbufs[j % NROWS],
                                      gsems[j % NROWS]).wait()

            def fire_scat(j):
                pltpu.async_copy(rbufs[j % NROWS],
                                 acc_sh.at[ibufs[j % NIDX].at[1]],
                                 ssems[j % NROWS], add=True)

            def drain_scat(j):
                pltpu.make_async_copy(rbufs[j % NROWS],
                                      acc_sh.at[ibufs[j % NIDX].at[1]],
                                      ssems[j % NROWS]).wait()

            for t in range(AGG_NCH + 4):
                if 0 <= t - 4 < AGG_NCH:
                    drain_scat(t - 4)
                if t < AGG_NCH:
                    fire_idx(t)
                if 0 <= t - 1 < AGG_NCH:
                    drain_idx(t - 1)
                    fire_rows(t - 1)
                if 0 <= t - 3 < AGG_NCH:
                    drain_rows(t - 3)
                    fire_scat(t - 3)
        return f

    pl.when(c == 0)(loop_for(y0))
    pl.when(c == 1)(loop_for(y1))
    plsc.subcore_barrier()

    def out_for(oref):
        def f():
            rows_copy(lambda o, n: acc_sh.at[pl.ds(o, n)],
                      lambda o, n: oref.at[pl.ds(o, n)])
        return f

    pl.when(c == 0)(out_for(o0))
    pl.when(c == 1)(out_for(o1))


BM = 400


def _mm_body(x_ref, w_ref, degt_ref, y0_ref, y1_ref):
    dis = lax.rsqrt(degt_ref[:, 0:1] + degt_ref[:, 1:2] + 1.0)
    xw = jnp.dot(x_ref[...], w_ref[...], preferred_element_type=jnp.float32)
    y = xw * dis
    y0_ref[...] = y[:, :H]
    y1_ref[...] = y[:, H:]


_mm_call = pl.pallas_call(
    _mm_body,
    grid=(N // BM,),
    in_specs=[
        pl.BlockSpec((BM, D), lambda i: (i, 0)),
        pl.BlockSpec((D, D), lambda i: (0, 0)),
        pl.BlockSpec((BM, 2), lambda i: (i, 0)),
    ],
    out_specs=(
        pl.BlockSpec((BM, H), lambda i: (i, 0)),
        pl.BlockSpec((BM, H), lambda i: (i, 0)),
    ),
    out_shape=(
        jax.ShapeDtypeStruct((N, H), jnp.float32),
        jax.ShapeDtypeStruct((N, H), jnp.float32),
    ),
)


def _epi_body(a0_ref, a1_ref, degt_ref, x_ref, b_ref, o_ref):
    dis = lax.rsqrt(degt_ref[:, 0:1] + degt_ref[:, 1:2] + 1.0)
    agg = jnp.concatenate([a0_ref[...], a1_ref[...]], axis=1)
    o_ref[...] = jnp.maximum(agg * dis + b_ref[...] + x_ref[...], 0.0)


_epi_call = pl.pallas_call(
    _epi_body,
    grid=(N // BM,),
    in_specs=[
        pl.BlockSpec((BM, H), lambda i: (i, 0)),
        pl.BlockSpec((BM, H), lambda i: (i, 0)),
        pl.BlockSpec((BM, 2), lambda i: (i, 0)),
        pl.BlockSpec((BM, D), lambda i: (i, 0)),
        pl.BlockSpec((1, D), lambda i: (0, 0)),
    ],
    out_specs=pl.BlockSpec((BM, D), lambda i: (i, 0)),
    out_shape=jax.ShapeDtypeStruct((N, D), jnp.float32),
)


def kernel(x, edge_index, W, b):
    src = edge_index[0].astype(jnp.int32)
    dst = edge_index[1].astype(jnp.int32)

    dst4 = dst.reshape(NC, NS, DEG_NCH, DEG_CHUNK)
    degp = _deg_kernel(dst4)
    degt = degp.T[:N]

    y0, y1 = _mm_call(x, W, degt)

    src3 = src.reshape(NS, AGG_NCH, AGG_CHUNK)
    dst3 = dst.reshape(NS, AGG_NCH, AGG_CHUNK)
    eidx = jnp.stack([src3, dst3], axis=2)
    a0, a1 = _agg_kernel(eidx, y0, y1)

    return _epi_call(a0, a1, degt, x, b.reshape(1, D))


# speedup vs baseline: 23.5527x; 1.0114x over previous
import functools

import jax
import jax.numpy as jnp
from jax import lax
from jax.experimental import pallas as pl
from jax.experimental.pallas import tpu as pltpu
from jax.experimental.pallas import tpu_sc as plsc

N = 10000
E = 160000
D = 256
H = D // 2
NC, NS = 2, 16
NPAD = 10240
RPAD = NPAD // NS
RTA = 632
RTB = N - 15 * RTA

_MESH = dict(core_axis_name="c", subcore_axis_name="s")

DEG_CHUNK = 125
DEG_EPW = E // (NC * NS)
DEG_NCH = DEG_EPW // DEG_CHUNK


@functools.partial(
    pl.kernel,
    out_type=jax.ShapeDtypeStruct((NC, NPAD), jnp.float32),
    mesh=plsc.VectorSubcoreMesh(**_MESH),
    scratch_types=[
        pltpu.VMEM((8, DEG_CHUNK), jnp.int32),
        pltpu.VMEM((128,), jnp.float32),
        pltpu.VMEM((RPAD,), jnp.float32),
        pltpu.VMEM_SHARED((NPAD,), jnp.float32),
    ],
)
def _deg_kernel(dst4, degp, dst_v, ones_v, zrow_v, deg_sh):
    c = lax.axis_index("c")
    s = lax.axis_index("s")
    for i in range(RPAD // 16):
        zrow_v[pl.ds(i * 16, 16)] = jnp.zeros((16,), jnp.float32)
    for i in range(8):
        ones_v[pl.ds(i * 16, 16)] = jnp.ones((16,), jnp.float32)

    pltpu.sync_copy(zrow_v, deg_sh.at[pl.ds(s * RPAD, RPAD)])
    plsc.subcore_barrier()

    for w in range(DEG_NCH // 8):
        pltpu.sync_copy(dst4.at[c, s, pl.ds(w * 8, 8)], dst_v)

        def step(j, carry):
            pltpu.sync_copy(
                ones_v.at[pl.ds(0, DEG_CHUNK)],
                deg_sh.at[dst_v.at[j]],
                add=True,
            )
            return carry

        lax.fori_loop(0, 8, step, 0)
    plsc.subcore_barrier()
    pltpu.sync_copy(
        deg_sh.at[pl.ds(s * RPAD, RPAD)],
        degp.at[c, pl.ds(s * RPAD, RPAD)],
    )


AGG_CHUNK = 80
AGG_EPT = E // NS
AGG_NCH = AGG_EPT // AGG_CHUNK
NROWS = 4
NIDX = 5


@functools.partial(
    pl.kernel,
    out_type=(
        jax.ShapeDtypeStruct((N, H), jnp.float32),
        jax.ShapeDtypeStruct((N, H), jnp.float32),
    ),
    mesh=plsc.VectorSubcoreMesh(**_MESH),
    scratch_types=(
        [pltpu.VMEM((2, AGG_CHUNK), jnp.int32) for _ in range(NIDX)]
        + [pltpu.VMEM((AGG_CHUNK, H), jnp.float32) for _ in range(NROWS)]
        + [pltpu.VMEM_SHARED((N, H), jnp.float32)]
        + [pltpu.SemaphoreType.DMA] * (NIDX + 2 * NROWS)
    ),
)
def _agg_kernel(eidx, y0, y1, o0, o1, *refs):
    ibufs = refs[:NIDX]
    rbufs = refs[NIDX:NIDX + NROWS]
    acc_sh = refs[NIDX + NROWS]
    isems = refs[NIDX + NROWS + 1:NIDX + NROWS + 1 + NIDX]
    gsems = refs[NIDX + NROWS + 1 + NIDX:NIDX + NROWS + 1 + NIDX + NROWS]
    ssems = refs[NIDX + NROWS + 1 + NIDX + NROWS:]
    c = lax.axis_index("c")
    s = lax.axis_index("s")
    r0 = s * RTA

    def rows_copy(get_src, get_dst):
        pltpu.sync_copy(get_src(r0, RTB), get_dst(r0, RTB))

        @pl.when(s < NS - 1)
        def _():
            pltpu.sync_copy(get_src(r0 + RTB, RTA - RTB),
                            get_dst(r0 + RTB, RTA - RTB))

    def init_for(yref):
        def f():
            rows_copy(lambda o, n: yref.at[pl.ds(o, n)],
                      lambda o, n: acc_sh.at[pl.ds(o, n)])
        return f

    pl.when(c == 0)(init_for(y0))
    pl.when(c == 1)(init_for(y1))
    plsc.subcore_barrier()

    def loop_for(yref):
        def f():
            def fire_idx(j):
                pltpu.async_copy(eidx.at[s, j], ibufs[j % NIDX],
                                 isems[j % NIDX])

            def drain_idx(j):
                pltpu.make_async_copy(eidx.at[s, 0], ibufs[j % NIDX],
                                      isems[j % NIDX]).wait()

            def fire_rows(j):
                pltpu.async_copy(yref.at[ibufs[j % NIDX].at[0]],
                                 rbufs[j % NROWS], gsems[j % NROWS])

            def drain_rows(j):
                pltpu.make_async_copy(yref.at[ibufs[j % NIDX].at[0]],
                                      rbufs[j % NROWS],
                                      gsems[j % NROWS]).wait()

            def fire_scat(j):
                pltpu.async_copy(rbufs[j % NROWS],
                                 acc_sh.at[ibufs[j % NIDX].at[1]],
                                 ssems[j % NROWS], add=True)

            def drain_scat(j):
                pltpu.make_async_copy(rbufs[j % NROWS],
                                      acc_sh.at[ibufs[j % NIDX].at[1]],
                                      ssems[j % NROWS]).wait()

            for t in range(AGG_NCH + NROWS + 1):
                if 0 <= t - NROWS - 1 < AGG_NCH:
                    drain_scat(t - NROWS - 1)
                if t < AGG_NCH:
                    fire_idx(t)
                if 0 <= t - 1 < AGG_NCH:
                    drain_idx(t - 1)
                    fire_rows(t - 1)
                if 0 <= t - NROWS < AGG_NCH:
                    drain_rows(t - NROWS)
                    fire_scat(t - NROWS)
        return f

    pl.when(c == 0)(loop_for(y0))
    pl.when(c == 1)(loop_for(y1))
    plsc.subcore_barrier()

    def out_for(oref):
        def f():
            rows_copy(lambda o, n: acc_sh.at[pl.ds(o, n)],
                      lambda o, n: oref.at[pl.ds(o, n)])
        return f

    pl.when(c == 0)(out_for(o0))
    pl.when(c == 1)(out_for(o1))


BM = 400


def _mm_body(x_ref, w_ref, degt_ref, y0_ref, y1_ref):
    dis = lax.rsqrt(degt_ref[:, 0:1] + degt_ref[:, 1:2] + 1.0)
    xw = jnp.dot(x_ref[...], w_ref[...], preferred_element_type=jnp.float32)
    y = xw * dis
    y0_ref[...] = y[:, :H]
    y1_ref[...] = y[:, H:]


_mm_call = pl.pallas_call(
    _mm_body,
    grid=(N // BM,),
    in_specs=[
        pl.BlockSpec((BM, D), lambda i: (i, 0)),
        pl.BlockSpec((D, D), lambda i: (0, 0)),
        pl.BlockSpec((BM, 2), lambda i: (i, 0)),
    ],
    out_specs=(
        pl.BlockSpec((BM, H), lambda i: (i, 0)),
        pl.BlockSpec((BM, H), lambda i: (i, 0)),
    ),
    out_shape=(
        jax.ShapeDtypeStruct((N, H), jnp.float32),
        jax.ShapeDtypeStruct((N, H), jnp.float32),
    ),
)


def _epi_body(a0_ref, a1_ref, degt_ref, x_ref, b_ref, o_ref):
    dis = lax.rsqrt(degt_ref[:, 0:1] + degt_ref[:, 1:2] + 1.0)
    agg = jnp.concatenate([a0_ref[...], a1_ref[...]], axis=1)
    o_ref[...] = jnp.maximum(agg * dis + b_ref[...] + x_ref[...], 0.0)


_epi_call = pl.pallas_call(
    _epi_body,
    grid=(N // BM,),
    in_specs=[
        pl.BlockSpec((BM, H), lambda i: (i, 0)),
        pl.BlockSpec((BM, H), lambda i: (i, 0)),
        pl.BlockSpec((BM, 2), lambda i: (i, 0)),
        pl.BlockSpec((BM, D), lambda i: (i, 0)),
        pl.BlockSpec((1, D), lambda i: (0, 0)),
    ],
    out_specs=pl.BlockSpec((BM, D), lambda i: (i, 0)),
    out_shape=jax.ShapeDtypeStruct((N, D), jnp.float32),
)


def kernel(x, edge_index, W, b):
    src = edge_index[0].astype(jnp.int32)
    dst = edge_index[1].astype(jnp.int32)

    dst4 = dst.reshape(NC, NS, DEG_NCH, DEG_CHUNK)
    degp = _deg_kernel(dst4)
    degt = degp.T[:N]

    y0, y1 = _mm_call(x, W, degt)

    src3 = src.reshape(NS, AGG_NCH, AGG_CHUNK)
    dst3 = dst.reshape(NS, AGG_NCH, AGG_CHUNK)
    eidx = jnp.stack([src3, dst3], axis=2)
    a0, a1 = _agg_kernel(eidx, y0, y1)

    return _epi_call(a0, a1, degt, x, b.reshape(1, D))
